# baseline (speedup 1.0000x reference)
import jax
import jax.numpy as jnp
from jax import lax
from jax.experimental import pallas as pl
from jax.experimental.pallas import tpu as pltpu
from jax.experimental.pallas import tpu_sc as plsc

N = 10000
E = 320000
D = 128
DH = D // 2
DE = 16
NCLS = 10

NC = 2
NS = 16
CK = 80
CHA = E // NS // CK
EPT = E // (NC * NS)
CHE = EPT // CK
E8 = E // 8
RPC = CK * DE // 128
NP = 10240
RPT = NP // NS
RZ = 64

_mesh = plsc.VectorSubcoreMesh(
    core_axis_name="c", subcore_axis_name="s", num_cores=NC, num_subcores=NS)


def _sc_aggmsg_body(hs_tbl, src3, dst3, outh,
                    idxs_v, idxd_v, rows_v, zb_v, acc_h, sg0, sg1, ss):
  cid = lax.axis_index("c")
  sid = lax.axis_index("s")
  z16 = jnp.zeros((16,), jnp.float32)
  sg = (sg0, sg1)

  @pl.loop(0, RZ)
  def _zfill(i):
    @pl.loop(0, D // 16)
    def _(c):
      zb_v[i, pl.ds(c * 16, 16)] = z16

  @pl.loop(0, RPT // RZ)
  def _zero_acc(r):
    pltpu.sync_copy(zb_v, acc_h.at[pl.ds(sid * RPT + r * RZ, RZ)])

  plsc.subcore_barrier()

  pltpu.sync_copy(src3.at[cid, sid], idxs_v)
  pltpu.sync_copy(dst3.at[cid, sid], idxd_v)

  for b in range(2):
    pltpu.async_copy(hs_tbl.at[idxs_v.at[b]], rows_v.at[b], sg[b])

  @pl.loop(0, CHE + 1, step=2)
  def _outer(j):
    for b in range(2):
      jj = j + b

      @pl.when(jj < CHE)
      def _(b=b, jj=jj):
        pltpu.make_async_copy(
            hs_tbl.at[idxs_v.at[jj]], rows_v.at[b], sg[b]).wait()
        dm = pltpu.async_copy(rows_v.at[b], acc_h.at[idxd_v.at[jj]], ss,
                              add=True)
        dm.wait()

        @pl.when(jj + 2 < CHE)
        def _():
          pltpu.async_copy(hs_tbl.at[idxs_v.at[jj + 2]], rows_v.at[b], sg[b])

  plsc.subcore_barrier()
  base = pl.multiple_of(sid * RPT, 8)
  pltpu.sync_copy(acc_h.at[pl.ds(base, RPT)], outh.at[cid, pl.ds(base, RPT)])


_sc_aggmsg = pl.kernel(
    _sc_aggmsg_body,
    out_type=jax.ShapeDtypeStruct((NC, NP, D), jnp.float32),
    mesh=_mesh,
    scratch_types=[
        pltpu.VMEM((CHE, CK), jnp.int32),
        pltpu.VMEM((CHE, CK), jnp.int32),
        pltpu.VMEM((2, CK, D), jnp.float32),
        pltpu.VMEM((RZ, D), jnp.float32),
        pltpu.VMEM_SHARED((NP, D), jnp.float32),
        pltpu.SemaphoreType.DMA,
        pltpu.SemaphoreType.DMA,
        pltpu.SemaphoreType.DMA,
    ],
    compiler_params=pltpu.CompilerParams(use_tc_tiling_on_sc=False),
)


def _sc_agg1f_body(hs_tbl, srco, dst3, ef4, outh, outxe, outxd,
                   idxs_v, idxd_v, rows_v, ef_v, ones_v, zb_v, zb16_v,
                   acc_h, acc_e, acc_d, sg0, sg1, ss, se0, se1):
  cid = lax.axis_index("c")
  sid = lax.axis_index("s")
  z16 = jnp.zeros((16,), jnp.float32)
  o16 = jnp.ones((16,), jnp.float32)
  sg = (sg0, sg1)
  se = (se0, se1)

  @pl.loop(0, RZ)
  def _zfill(i):
    zb16_v[i, :] = z16
    @pl.loop(0, DH // 16)
    def _(c):
      zb_v[i, pl.ds(c * 16, 16)] = z16

  @pl.loop(0, CK)
  def _ofill(i):
    ones_v[i, :] = o16

  @pl.loop(0, RPT // RZ)
  def _zero_acc(r):
    pltpu.sync_copy(zb_v, acc_h.at[pl.ds(sid * RPT + r * RZ, RZ)])
    pltpu.sync_copy(zb16_v, acc_e.at[pl.ds(sid * RPT + r * RZ, RZ)])
    pltpu.sync_copy(zb16_v, acc_d.at[pl.ds(sid * RPT + r * RZ, RZ)])

  plsc.subcore_barrier()

  pltpu.sync_copy(srco.at[cid, sid], idxs_v)
  pltpu.sync_copy(dst3.at[sid], idxd_v)

  for b in range(2):
    pltpu.async_copy(hs_tbl.at[idxs_v.at[b]], rows_v.at[b], sg[b])

    @pl.when(cid == b)
    def _(b=b):
      pltpu.async_copy(ef4.at[sid, b], ef_v.at[b], se[b])

  @pl.loop(0, CHA, step=2)
  def _outer(j):
    for b in range(2):
      jj = j + b
      pltpu.make_async_copy(
          hs_tbl.at[idxs_v.at[jj]], rows_v.at[b], sg[b]).wait()
      dm = pltpu.async_copy(rows_v.at[b], acc_h.at[idxd_v.at[jj]], ss,
                            add=True)

      @pl.when(cid == b)
      def _(b=b, jj=jj):
        pltpu.make_async_copy(ef4.at[sid, jj], ef_v.at[b], se[b]).wait()
        pltpu.sync_copy(ef_v.at[b], acc_e.at[idxd_v.at[jj]], add=True)

        @pl.when(jj + 2 < CHA)
        def _():
          pltpu.async_copy(ef4.at[sid, jj + 2], ef_v.at[b], se[b])

      @pl.when(cid != b)
      def _(jj=jj):
        pltpu.sync_copy(ones_v, acc_d.at[idxd_v.at[jj]], add=True)

      dm.wait()

      @pl.when(jj + 2 < CHA)
      def _(b=b, jj=jj):
        pltpu.async_copy(hs_tbl.at[idxs_v.at[jj + 2]], rows_v.at[b], sg[b])

  plsc.subcore_barrier()
  base = pl.multiple_of(sid * RPT, 8)
  pltpu.sync_copy(acc_h.at[pl.ds(base, RPT)], outh.at[cid, pl.ds(base, RPT)])
  pltpu.sync_copy(acc_e.at[pl.ds(base, RPT)], outxe.at[cid, pl.ds(base, RPT)])
  pltpu.sync_copy(acc_d.at[pl.ds(base, RPT)], outxd.at[cid, pl.ds(base, RPT)])


_sc_agg1f = pl.kernel(
    _sc_agg1f_body,
    out_type=[jax.ShapeDtypeStruct((NC, NP, DH), jnp.float32),
              jax.ShapeDtypeStruct((NC, NP, DE), jnp.float32),
              jax.ShapeDtypeStruct((NC, NP, DE), jnp.float32)],
    mesh=_mesh,
    scratch_types=[
        pltpu.VMEM((CHA, CK), jnp.int32),
        pltpu.VMEM((CHA, CK), jnp.int32),
        pltpu.VMEM((2, CK, DH), jnp.float32),
        pltpu.VMEM((2, CK, DE), jnp.float32),
        pltpu.VMEM((CK, DE), jnp.float32),
        pltpu.VMEM((RZ, DH), jnp.float32),
        pltpu.VMEM((RZ, DE), jnp.float32),
        pltpu.VMEM_SHARED((NP, DH), jnp.float32),
        pltpu.VMEM_SHARED((NP, DE), jnp.float32),
        pltpu.VMEM_SHARED((NP, DE), jnp.float32),
        pltpu.SemaphoreType.DMA,
        pltpu.SemaphoreType.DMA,
        pltpu.SemaphoreType.DMA,
        pltpu.SemaphoreType.DMA,
        pltpu.SemaphoreType.DMA,
    ],
    compiler_params=pltpu.CompilerParams(use_tc_tiling_on_sc=False),
)


def _sc_aggx_body(ef4, dst3, outx, idxd_v, ef_v, ones_v, zb16_v, acc_x,
                  se0, se1):
  cid = lax.axis_index("c")
  sid = lax.axis_index("s")
  z16 = jnp.zeros((16,), jnp.float32)
  o16 = jnp.ones((16,), jnp.float32)
  se = (se0, se1)

  @pl.loop(0, RZ)
  def _zfill(i):
    zb16_v[i, :] = z16

  @pl.loop(0, CK)
  def _ofill(i):
    ones_v[i, :] = o16

  @pl.loop(0, RPT // RZ)
  def _zero_acc(r):
    pltpu.sync_copy(zb16_v, acc_x.at[pl.ds(sid * RPT + r * RZ, RZ)])

  plsc.subcore_barrier()

  pltpu.sync_copy(dst3.at[sid], idxd_v)

  for b in range(2):
    @pl.when(cid == 0)
    def _(b=b):
      pltpu.async_copy(ef4.at[sid, b], ef_v.at[b], se[b])

  @pl.loop(0, CHA, step=2)
  def _outer(j):
    for b in range(2):
      jj = j + b

      @pl.when(cid == 0)
      def _(b=b, jj=jj):
        pltpu.make_async_copy(ef4.at[sid, jj], ef_v.at[b], se[b]).wait()
        pltpu.sync_copy(ef_v.at[b], acc_x.at[idxd_v.at[jj]], add=True)

        @pl.when(jj + 2 < CHA)
        def _():
          pltpu.async_copy(ef4.at[sid, jj + 2], ef_v.at[b], se[b])

      @pl.when(cid == 1)
      def _(jj=jj):
        pltpu.sync_copy(ones_v, acc_x.at[idxd_v.at[jj]], add=True)

  plsc.subcore_barrier()
  base = pl.multiple_of(sid * RPT, 8)
  pltpu.sync_copy(acc_x.at[pl.ds(base, RPT)], outx.at[cid, pl.ds(base, RPT)])


_sc_aggx = pl.kernel(
    _sc_aggx_body,
    out_type=jax.ShapeDtypeStruct((NC, NP, DE), jnp.float32),
    mesh=_mesh,
    scratch_types=[
        pltpu.VMEM((CHA, CK), jnp.int32),
        pltpu.VMEM((2, CK, DE), jnp.float32),
        pltpu.VMEM((CK, DE), jnp.float32),
        pltpu.VMEM((RZ, DE), jnp.float32),
        pltpu.VMEM_SHARED((NP, DE), jnp.float32),
        pltpu.SemaphoreType.DMA,
        pltpu.SemaphoreType.DMA,
    ],
    compiler_params=pltpu.CompilerParams(use_tc_tiling_on_sc=False),
)


def _sc_edge_body(pu, pv, src3, dst3, outs, idxs_v, idxd_v, a_v, b_v, o_v,
                  sa0, sa1, sb0, sb1, so0, so1):
  cid = lax.axis_index("c")
  sid = lax.axis_index("s")
  sa = (sa0, sa1)
  sb = (sb0, sb1)
  so = (so0, so1)
  pltpu.sync_copy(src3.at[cid, sid], idxs_v)
  pltpu.sync_copy(dst3.at[cid, sid], idxd_v)
  rbase = (cid * NS + sid) * (EPT // 8)

  for b in range(2):
    pltpu.async_copy(pu.at[idxs_v.at[b]], a_v.at[b], sa[b])
    pltpu.async_copy(pv.at[idxd_v.at[b]], b_v.at[b], sb[b])

  @pl.loop(0, CHE + 1, step=2)
  def _outer(j):
    for b in range(2):
      jj = j + b

      @pl.when(jj < CHE)
      def _(b=b, jj=jj):
        pltpu.make_async_copy(pu.at[idxs_v.at[jj]], a_v.at[b], sa[b]).wait()
        pltpu.make_async_copy(pv.at[idxd_v.at[jj]], b_v.at[b], sb[b]).wait()

        @pl.when(jj >= 2)
        def _():
          pltpu.make_async_copy(
              o_v.at[b], outs.at[pl.ds(rbase + (jj - 2) * RPC, RPC)],
              so[b]).wait()

        for i in range(CK):
          o_v[b, i // 8, pl.ds((i % 8) * DE, DE)] = a_v[b, i, :] + b_v[b, i, :]

        pltpu.async_copy(o_v.at[b], outs.at[pl.ds(rbase + jj * RPC, RPC)],
                         so[b])

        @pl.when(jj + 2 < CHE)
        def _():
          pltpu.async_copy(pu.at[idxs_v.at[jj + 2]], a_v.at[b], sa[b])
          pltpu.async_copy(pv.at[idxd_v.at[jj + 2]], b_v.at[b], sb[b])

  for b, jl in ((1, CHE - 2), (0, CHE - 1)):
    pltpu.make_async_copy(
        o_v.at[b], outs.at[pl.ds(rbase + jl * RPC, RPC)], so[b]).wait()


_sc_edge = pl.kernel(
    _sc_edge_body,
    out_type=jax.ShapeDtypeStruct((E8, 128), jnp.float32),
    mesh=_mesh,
    scratch_types=[
        pltpu.VMEM((CHE, CK), jnp.int32),
        pltpu.VMEM((CHE, CK), jnp.int32),
        pltpu.VMEM((2, CK, DE), jnp.float32),
        pltpu.VMEM((2, CK, DE), jnp.float32),
        pltpu.VMEM((2, RPC, 128), jnp.float32),
        pltpu.SemaphoreType.DMA,
        pltpu.SemaphoreType.DMA,
        pltpu.SemaphoreType.DMA,
        pltpu.SemaphoreType.DMA,
        pltpu.SemaphoreType.DMA,
        pltpu.SemaphoreType.DMA,
    ],
    compiler_params=pltpu.CompilerParams(use_tc_tiling_on_sc=False),
)


BLK = 2000


def _mm_body(x_ref, w_ref, o_ref):
  r = jnp.dot(x_ref[...], w_ref[...], preferred_element_type=jnp.float32)
  o_ref[0] = r[:, :DH]
  o_ref[1] = r[:, DH:]


_tc_mm = pl.pallas_call(
    _mm_body,
    grid=(N // BLK,),
    in_specs=[pl.BlockSpec((BLK, D), lambda i: (i, 0)),
              pl.BlockSpec((D, D), lambda i: (0, 0))],
    out_specs=pl.BlockSpec((NC, BLK, DH), lambda i: (0, i, 0)),
    out_shape=jax.ShapeDtypeStruct((NC, N, DH), jnp.float32),
)


def _tc_mid_body(ph, pxe, pxd, h, we, bm, was, wan, ba, wm2, h1o, hs2o):
  aggh = jnp.concatenate([ph[0], ph[1]], axis=1)
  agge = pxe[0] + pxe[1]
  deg = (pxd[0] + pxd[1])[:, 0:1]
  neigh = (aggh + jnp.dot(agge, we[...], preferred_element_type=jnp.float32)
           + deg * bm[...])
  neigh = neigh / jnp.maximum(deg, 1.0)
  h1 = jnp.maximum(
      jnp.dot(h[...], was[...], preferred_element_type=jnp.float32)
      + jnp.dot(neigh, wan[...], preferred_element_type=jnp.float32)
      + ba[...], 0.0)
  h1o[...] = h1
  hs2o[...] = jnp.dot(h1, wm2[...], preferred_element_type=jnp.float32)


_tc_mid = pl.pallas_call(
    _tc_mid_body,
    grid=(N // BLK,),
    in_specs=[pl.BlockSpec((NC, BLK, DH), lambda i: (0, i, 0)),
              pl.BlockSpec((NC, BLK, DE), lambda i: (0, i, 0)),
              pl.BlockSpec((NC, BLK, DE), lambda i: (0, i, 0)),
              pl.BlockSpec((BLK, D), lambda i: (i, 0)),
              pl.BlockSpec((DE, D), lambda i: (0, 0)),
              pl.BlockSpec((1, D), lambda i: (0, 0)),
              pl.BlockSpec((D, D), lambda i: (0, 0)),
              pl.BlockSpec((D, D), lambda i: (0, 0)),
              pl.BlockSpec((1, D), lambda i: (0, 0)),
              pl.BlockSpec((D, D), lambda i: (0, 0))],
    out_specs=[pl.BlockSpec((BLK, D), lambda i: (i, 0)),
               pl.BlockSpec((BLK, D), lambda i: (i, 0))],
    out_shape=[jax.ShapeDtypeStruct((N, D), jnp.float32),
               jax.ShapeDtypeStruct((N, D), jnp.float32)],
)


def _tc_fin_body(ph, pxe, pxd, h1, we, bm, was, wan, ba, wu, wv, bp,
                 puo, pvo):
  aggh = ph[0] + ph[1]
  agge = pxe[0] + pxe[1]
  deg = (pxd[0] + pxd[1])[:, 0:1]
  neigh = (aggh + jnp.dot(agge, we[...], preferred_element_type=jnp.float32)
           + deg * bm[...])
  neigh = neigh / jnp.maximum(deg, 1.0)
  h2 = jnp.maximum(
      jnp.dot(h1[...], was[...], preferred_element_type=jnp.float32)
      + jnp.dot(neigh, wan[...], preferred_element_type=jnp.float32)
      + ba[...], 0.0)
  puo[...] = jnp.dot(h2, wu[...], preferred_element_type=jnp.float32) + bp[...]
  pvo[...] = jnp.dot(h2, wv[...], preferred_element_type=jnp.float32)


_tc_fin = pl.pallas_call(
    _tc_fin_body,
    grid=(N // BLK,),
    in_specs=[pl.BlockSpec((NC, BLK, D), lambda i: (0, i, 0)),
              pl.BlockSpec((NC, BLK, DE), lambda i: (0, i, 0)),
              pl.BlockSpec((NC, BLK, DE), lambda i: (0, i, 0)),
              pl.BlockSpec((BLK, D), lambda i: (i, 0)),
              pl.BlockSpec((DE, D), lambda i: (0, 0)),
              pl.BlockSpec((1, D), lambda i: (0, 0)),
              pl.BlockSpec((D, D), lambda i: (0, 0)),
              pl.BlockSpec((D, D), lambda i: (0, 0)),
              pl.BlockSpec((1, D), lambda i: (0, 0)),
              pl.BlockSpec((D, DE), lambda i: (0, 0)),
              pl.BlockSpec((D, DE), lambda i: (0, 0)),
              pl.BlockSpec((1, DE), lambda i: (0, 0))],
    out_specs=[pl.BlockSpec((BLK, DE), lambda i: (i, 0)),
               pl.BlockSpec((BLK, DE), lambda i: (i, 0))],
    out_shape=[jax.ShapeDtypeStruct((N, DE), jnp.float32),
               jax.ShapeDtypeStruct((N, DE), jnp.float32)],
)


def kernel(nfeats, efeats, edge_index, W_msg1, b_msg1, W_apply1, b_apply1,
           W_msg2, b_msg2, W_apply2, b_apply2, W_pred, b_pred):
  nf = nfeats.reshape(N, D)
  src = edge_index[0].astype(jnp.int32)
  dst = edge_index[1].astype(jnp.int32)
  srco = (src.reshape(1, NS, CHA, CK)
          + (jnp.arange(NC, dtype=jnp.int32) * N).reshape(NC, 1, 1, 1))
  dstA = dst.reshape(NS, CHA, CK)
  ef4 = efeats.reshape(NS, CHA, CK, DE)
  srcE = src.reshape(NC, NS, CHE, CK)
  dstE = dst.reshape(NC, NS, CHE, CK)

  hs1 = _tc_mm(nf, W_msg1[:D]).reshape(NC * N, DH)
  ph1, pxe, pxd = _sc_agg1f(hs1, srco, dstA, ef4)
  h1, hs2 = _tc_mid(ph1, pxe, pxd, nf, W_msg1[D:], b_msg1.reshape(1, D),
                    W_apply1[:D], W_apply1[D:], b_apply1.reshape(1, D),
                    W_msg2[:D])
  ph2 = _sc_aggmsg(hs2, srcE, dstE)
  wu = jnp.pad(W_pred[:D], ((0, 0), (0, DE - NCLS)))
  wv = jnp.pad(W_pred[D:], ((0, 0), (0, DE - NCLS)))
  bp = jnp.pad(b_pred, (0, DE - NCLS)).reshape(1, DE)
  pu16, pv16 = _tc_fin(ph2, pxe, pxd, h1, W_msg2[D:], b_msg2.reshape(1, D),
                       W_apply2[:D], W_apply2[D:], b_apply2.reshape(1, D),
                       wu, wv, bp)
  s8 = _sc_edge(pu16, pv16, srcE, dstE)
  return s8.reshape(E, DE)[:, :NCLS]


# speedup vs baseline: 1.0177x; 1.0177x over previous
import jax
import jax.numpy as jnp
from jax import lax
from jax.experimental import pallas as pl
from jax.experimental.pallas import tpu as pltpu
from jax.experimental.pallas import tpu_sc as plsc

N = 10000
E = 320000
D = 128
DH = D // 2
DE = 16
NCLS = 10

NC = 2
NS = 16
CK = 80
CHA = E // NS // CK
EPT = E // (NC * NS)
CHE = EPT // CK
E8 = E // 8
RPC = CK * DE // 128
NP = 10240
RPT = NP // NS
RZ = 64

_mesh = plsc.VectorSubcoreMesh(
    core_axis_name="c", subcore_axis_name="s", num_cores=NC, num_subcores=NS)


def _sc_aggmsg_body(hs_tbl, src3, dst3, outh,
                    idxs_v, idxd_v, rows_v, zb_v, acc_h, sg0, sg1, ss):
  cid = lax.axis_index("c")
  sid = lax.axis_index("s")
  z16 = jnp.zeros((16,), jnp.float32)
  sg = (sg0, sg1)

  @pl.loop(0, RZ)
  def _zfill(i):
    @pl.loop(0, D // 16)
    def _(c):
      zb_v[i, pl.ds(c * 16, 16)] = z16

  @pl.loop(0, RPT // RZ)
  def _zero_acc(r):
    pltpu.sync_copy(zb_v, acc_h.at[pl.ds(sid * RPT + r * RZ, RZ)])

  plsc.subcore_barrier()

  pltpu.sync_copy(src3.at[cid, sid], idxs_v)
  pltpu.sync_copy(dst3.at[cid, sid], idxd_v)

  for b in range(2):
    pltpu.async_copy(hs_tbl.at[idxs_v.at[b]], rows_v.at[b], sg[b])

  @pl.loop(0, CHE + 1, step=2)
  def _outer(j):
    for b in range(2):
      jj = j + b

      @pl.when(jj < CHE)
      def _(b=b, jj=jj):
        pltpu.make_async_copy(
            hs_tbl.at[idxs_v.at[jj]], rows_v.at[b], sg[b]).wait()
        dm = pltpu.async_copy(rows_v.at[b], acc_h.at[idxd_v.at[jj]], ss,
                              add=True)
        dm.wait()

        @pl.when(jj + 2 < CHE)
        def _():
          pltpu.async_copy(hs_tbl.at[idxs_v.at[jj + 2]], rows_v.at[b], sg[b])

  plsc.subcore_barrier()
  base = pl.multiple_of(sid * RPT, 8)
  pltpu.sync_copy(acc_h.at[pl.ds(base, RPT)], outh.at[cid, pl.ds(base, RPT)])


_sc_aggmsg = pl.kernel(
    _sc_aggmsg_body,
    out_type=jax.ShapeDtypeStruct((NC, NP, D), jnp.float32),
    mesh=_mesh,
    scratch_types=[
        pltpu.VMEM((CHE, CK), jnp.int32),
        pltpu.VMEM((CHE, CK), jnp.int32),
        pltpu.VMEM((2, CK, D), jnp.float32),
        pltpu.VMEM((RZ, D), jnp.float32),
        pltpu.VMEM_SHARED((NP, D), jnp.float32),
        pltpu.SemaphoreType.DMA,
        pltpu.SemaphoreType.DMA,
        pltpu.SemaphoreType.DMA,
    ],
    compiler_params=pltpu.CompilerParams(use_tc_tiling_on_sc=False),
)


def _sc_agg1f_body(hs_tbl, srco, dst3, ef4, outh, outx,
                   idxs_v, idxd_v, rows_v, ef_v, ones_v, zb_v, zb16_v,
                   acc_h, acc_x, sg0, sg1, ss, se0, se1):
  cid = lax.axis_index("c")
  sid = lax.axis_index("s")
  z16 = jnp.zeros((16,), jnp.float32)
  o16 = jnp.ones((16,), jnp.float32)
  sg = (sg0, sg1)
  se = (se0, se1)

  @pl.loop(0, RZ)
  def _zfill(i):
    zb16_v[i, :] = z16
    @pl.loop(0, DH // 16)
    def _(c):
      zb_v[i, pl.ds(c * 16, 16)] = z16

  @pl.loop(0, CK)
  def _ofill(i):
    ones_v[i, :] = o16

  @pl.loop(0, RPT // RZ)
  def _zero_acc(r):
    pltpu.sync_copy(zb_v, acc_h.at[pl.ds(sid * RPT + r * RZ, RZ)])
    pltpu.sync_copy(zb16_v, acc_x.at[pl.ds(sid * RPT + r * RZ, RZ)])

  plsc.subcore_barrier()

  pltpu.sync_copy(srco.at[cid, sid], idxs_v)
  pltpu.sync_copy(dst3.at[sid], idxd_v)

  for b in range(2):
    pltpu.async_copy(hs_tbl.at[idxs_v.at[b]], rows_v.at[b], sg[b])

    @pl.when(cid == 0)
    def _(b=b):
      pltpu.async_copy(ef4.at[sid, b], ef_v.at[b], se[b])

  @pl.loop(0, CHA, step=2)
  def _outer(j):
    for b in range(2):
      jj = j + b
      pltpu.make_async_copy(
          hs_tbl.at[idxs_v.at[jj]], rows_v.at[b], sg[b]).wait()
      dm = pltpu.async_copy(rows_v.at[b], acc_h.at[idxd_v.at[jj]], ss,
                            add=True)

      @pl.when(cid == 0)
      def _(b=b, jj=jj):
        pltpu.make_async_copy(ef4.at[sid, jj], ef_v.at[b], se[b]).wait()
        pltpu.sync_copy(ef_v.at[b], acc_x.at[idxd_v.at[jj]], add=True)

        @pl.when(jj + 2 < CHA)
        def _():
          pltpu.async_copy(ef4.at[sid, jj + 2], ef_v.at[b], se[b])

      @pl.when(cid == 1)
      def _(jj=jj):
        pltpu.sync_copy(ones_v, acc_x.at[idxd_v.at[jj]], add=True)

      dm.wait()

      @pl.when(jj + 2 < CHA)
      def _(b=b, jj=jj):
        pltpu.async_copy(hs_tbl.at[idxs_v.at[jj + 2]], rows_v.at[b], sg[b])

  plsc.subcore_barrier()
  base = pl.multiple_of(sid * RPT, 8)
  pltpu.sync_copy(acc_h.at[pl.ds(base, RPT)], outh.at[cid, pl.ds(base, RPT)])
  pltpu.sync_copy(acc_x.at[pl.ds(base, RPT)], outx.at[cid, pl.ds(base, RPT)])


_sc_agg1f = pl.kernel(
    _sc_agg1f_body,
    out_type=[jax.ShapeDtypeStruct((NC, NP, DH), jnp.float32),
              jax.ShapeDtypeStruct((NC, NP, DE), jnp.float32)],
    mesh=_mesh,
    scratch_types=[
        pltpu.VMEM((CHA, CK), jnp.int32),
        pltpu.VMEM((CHA, CK), jnp.int32),
        pltpu.VMEM((2, CK, DH), jnp.float32),
        pltpu.VMEM((2, CK, DE), jnp.float32),
        pltpu.VMEM((CK, DE), jnp.float32),
        pltpu.VMEM((RZ, DH), jnp.float32),
        pltpu.VMEM((RZ, DE), jnp.float32),
        pltpu.VMEM_SHARED((NP, DH), jnp.float32),
        pltpu.VMEM_SHARED((NP, DE), jnp.float32),
        pltpu.SemaphoreType.DMA,
        pltpu.SemaphoreType.DMA,
        pltpu.SemaphoreType.DMA,
        pltpu.SemaphoreType.DMA,
        pltpu.SemaphoreType.DMA,
    ],
    compiler_params=pltpu.CompilerParams(use_tc_tiling_on_sc=False),
)


def _sc_edge_body(pu, pv, src3, dst3, outs, idxs_v, idxd_v, a_v, b_v, o_v,
                  sa0, sa1, sb0, sb1, so0, so1):
  cid = lax.axis_index("c")
  sid = lax.axis_index("s")
  sa = (sa0, sa1)
  sb = (sb0, sb1)
  so = (so0, so1)
  pltpu.sync_copy(src3.at[cid, sid], idxs_v)
  pltpu.sync_copy(dst3.at[cid, sid], idxd_v)
  rbase = (cid * NS + sid) * (EPT // 8)

  for b in range(2):
    pltpu.async_copy(pu.at[idxs_v.at[b]], a_v.at[b], sa[b])
    pltpu.async_copy(pv.at[idxd_v.at[b]], b_v.at[b], sb[b])

  @pl.loop(0, CHE + 1, step=2)
  def _outer(j):
    for b in range(2):
      jj = j + b

      @pl.when(jj < CHE)
      def _(b=b, jj=jj):
        pltpu.make_async_copy(pu.at[idxs_v.at[jj]], a_v.at[b], sa[b]).wait()
        pltpu.make_async_copy(pv.at[idxd_v.at[jj]], b_v.at[b], sb[b]).wait()

        @pl.when(jj >= 2)
        def _():
          pltpu.make_async_copy(
              o_v.at[b], outs.at[pl.ds(rbase + (jj - 2) * RPC, RPC)],
              so[b]).wait()

        for i in range(CK):
          o_v[b, i // 8, pl.ds((i % 8) * DE, DE)] = a_v[b, i, :] + b_v[b, i, :]

        pltpu.async_copy(o_v.at[b], outs.at[pl.ds(rbase + jj * RPC, RPC)],
                         so[b])

        @pl.when(jj + 2 < CHE)
        def _():
          pltpu.async_copy(pu.at[idxs_v.at[jj + 2]], a_v.at[b], sa[b])
          pltpu.async_copy(pv.at[idxd_v.at[jj + 2]], b_v.at[b], sb[b])

  for b, jl in ((1, CHE - 2), (0, CHE - 1)):
    pltpu.make_async_copy(
        o_v.at[b], outs.at[pl.ds(rbase + jl * RPC, RPC)], so[b]).wait()


_sc_edge = pl.kernel(
    _sc_edge_body,
    out_type=jax.ShapeDtypeStruct((E8, 128), jnp.float32),
    mesh=_mesh,
    scratch_types=[
        pltpu.VMEM((CHE, CK), jnp.int32),
        pltpu.VMEM((CHE, CK), jnp.int32),
        pltpu.VMEM((2, CK, DE), jnp.float32),
        pltpu.VMEM((2, CK, DE), jnp.float32),
        pltpu.VMEM((2, RPC, 128), jnp.float32),
        pltpu.SemaphoreType.DMA,
        pltpu.SemaphoreType.DMA,
        pltpu.SemaphoreType.DMA,
        pltpu.SemaphoreType.DMA,
        pltpu.SemaphoreType.DMA,
        pltpu.SemaphoreType.DMA,
    ],
    compiler_params=pltpu.CompilerParams(use_tc_tiling_on_sc=False),
)


BLK = 2000


def _mm_body(x_ref, w_ref, o_ref):
  r = jnp.dot(x_ref[...], w_ref[...], preferred_element_type=jnp.float32)
  o_ref[0] = r[:, :DH]
  o_ref[1] = r[:, DH:]


_tc_mm = pl.pallas_call(
    _mm_body,
    grid=(N // BLK,),
    in_specs=[pl.BlockSpec((BLK, D), lambda i: (i, 0)),
              pl.BlockSpec((D, D), lambda i: (0, 0))],
    out_specs=pl.BlockSpec((NC, BLK, DH), lambda i: (0, i, 0)),
    out_shape=jax.ShapeDtypeStruct((NC, N, DH), jnp.float32),
)


def _tc_mid_body(ph, px, h, we, bm, was, wan, ba, wm2, h1o, hs2o):
  aggh = jnp.concatenate([ph[0], ph[1]], axis=1)
  agge = px[0]
  deg = px[1][:, 0:1]
  neigh = (aggh + jnp.dot(agge, we[...], preferred_element_type=jnp.float32)
           + deg * bm[...])
  neigh = neigh / jnp.maximum(deg, 1.0)
  h1 = jnp.maximum(
      jnp.dot(h[...], was[...], preferred_element_type=jnp.float32)
      + jnp.dot(neigh, wan[...], preferred_element_type=jnp.float32)
      + ba[...], 0.0)
  h1o[...] = h1
  hs2o[...] = jnp.dot(h1, wm2[...], preferred_element_type=jnp.float32)


_tc_mid = pl.pallas_call(
    _tc_mid_body,
    grid=(N // BLK,),
    in_specs=[pl.BlockSpec((NC, BLK, DH), lambda i: (0, i, 0)),
              pl.BlockSpec((NC, BLK, DE), lambda i: (0, i, 0)),
              pl.BlockSpec((BLK, D), lambda i: (i, 0)),
              pl.BlockSpec((DE, D), lambda i: (0, 0)),
              pl.BlockSpec((1, D), lambda i: (0, 0)),
              pl.BlockSpec((D, D), lambda i: (0, 0)),
              pl.BlockSpec((D, D), lambda i: (0, 0)),
              pl.BlockSpec((1, D), lambda i: (0, 0)),
              pl.BlockSpec((D, D), lambda i: (0, 0))],
    out_specs=[pl.BlockSpec((BLK, D), lambda i: (i, 0)),
               pl.BlockSpec((BLK, D), lambda i: (i, 0))],
    out_shape=[jax.ShapeDtypeStruct((N, D), jnp.float32),
               jax.ShapeDtypeStruct((N, D), jnp.float32)],
)


def _tc_fin_body(ph, px, h1, we, bm, was, wan, ba, wu, wv, bp, puo, pvo):
  aggh = ph[0] + ph[1]
  agge = px[0]
  deg = px[1][:, 0:1]
  neigh = (aggh + jnp.dot(agge, we[...], preferred_element_type=jnp.float32)
           + deg * bm[...])
  neigh = neigh / jnp.maximum(deg, 1.0)
  h2 = jnp.maximum(
      jnp.dot(h1[...], was[...], preferred_element_type=jnp.float32)
      + jnp.dot(neigh, wan[...], preferred_element_type=jnp.float32)
      + ba[...], 0.0)
  puo[...] = jnp.dot(h2, wu[...], preferred_element_type=jnp.float32) + bp[...]
  pvo[...] = jnp.dot(h2, wv[...], preferred_element_type=jnp.float32)


_tc_fin = pl.pallas_call(
    _tc_fin_body,
    grid=(N // BLK,),
    in_specs=[pl.BlockSpec((NC, BLK, D), lambda i: (0, i, 0)),
              pl.BlockSpec((NC, BLK, DE), lambda i: (0, i, 0)),
              pl.BlockSpec((BLK, D), lambda i: (i, 0)),
              pl.BlockSpec((DE, D), lambda i: (0, 0)),
              pl.BlockSpec((1, D), lambda i: (0, 0)),
              pl.BlockSpec((D, D), lambda i: (0, 0)),
              pl.BlockSpec((D, D), lambda i: (0, 0)),
              pl.BlockSpec((1, D), lambda i: (0, 0)),
              pl.BlockSpec((D, DE), lambda i: (0, 0)),
              pl.BlockSpec((D, DE), lambda i: (0, 0)),
              pl.BlockSpec((1, DE), lambda i: (0, 0))],
    out_specs=[pl.BlockSpec((BLK, DE), lambda i: (i, 0)),
               pl.BlockSpec((BLK, DE), lambda i: (i, 0))],
    out_shape=[jax.ShapeDtypeStruct((N, DE), jnp.float32),
               jax.ShapeDtypeStruct((N, DE), jnp.float32)],
)


def kernel(nfeats, efeats, edge_index, W_msg1, b_msg1, W_apply1, b_apply1,
           W_msg2, b_msg2, W_apply2, b_apply2, W_pred, b_pred):
  nf = nfeats.reshape(N, D)
  src = edge_index[0].astype(jnp.int32)
  dst = edge_index[1].astype(jnp.int32)
  srco = (src.reshape(1, NS, CHA, CK)
          + (jnp.arange(NC, dtype=jnp.int32) * N).reshape(NC, 1, 1, 1))
  dstA = dst.reshape(NS, CHA, CK)
  ef4 = efeats.reshape(NS, CHA, CK, DE)
  srcE = src.reshape(NC, NS, CHE, CK)
  dstE = dst.reshape(NC, NS, CHE, CK)

  hs1 = _tc_mm(nf, W_msg1[:D]).reshape(NC * N, DH)
  ph1, px = _sc_agg1f(hs1, srco, dstA, ef4)
  h1, hs2 = _tc_mid(ph1, px, nf, W_msg1[D:], b_msg1.reshape(1, D),
                    W_apply1[:D], W_apply1[D:], b_apply1.reshape(1, D),
                    W_msg2[:D])
  ph2 = _sc_aggmsg(hs2, srcE, dstE)
  wu = jnp.pad(W_pred[:D], ((0, 0), (0, DE - NCLS)))
  wv = jnp.pad(W_pred[D:], ((0, 0), (0, DE - NCLS)))
  bp = jnp.pad(b_pred, (0, DE - NCLS)).reshape(1, DE)
  pu16, pv16 = _tc_fin(ph2, px, h1, W_msg2[D:], b_msg2.reshape(1, D),
                       W_apply2[:D], W_apply2[D:], b_apply2.reshape(1, D),
                       wu, wv, bp)
  s8 = _sc_edge(pu16, pv16, srcE, dstE)
  return s8.reshape(E, DE)[:, :NCLS]
